# baseline (speedup 1.0000x reference)
import functools

import jax
import jax.numpy as jnp
from jax import lax
from jax.experimental import pallas as pl
from jax.experimental.pallas import tpu as pltpu
from jax.experimental.pallas import tpu_sc as plsc

OUT_H, OUT_W = 448, 448
B_TOTAL = 128
H, W = 512, 512

NC, NS = 2, 16
NW = NC * NS
SPW = B_TOTAL // NW
R = 16
N_CH = OUT_H // R
KV = OUT_W // 16
IN_W = OUT_W + 8


def _offsets(h_range, w_range):
    kk = jax.random.key(42)
    kx, ky = jax.random.split(kk)
    xo = jax.random.randint(kx, (B_TOTAL,), 0, h_range, dtype=jnp.int32)
    yo = jax.random.randint(ky, (B_TOTAL,), 0, w_range, dtype=jnp.int32)
    return xo, yo


def _sc_body(x2, offs, out2, off_v, in_buf, out_buf):
    c = lax.axis_index("c")
    s = lax.axis_index("s")
    wid = s * NC + c
    pltpu.sync_copy(offs.at[wid], off_v)
    ovec = off_v[...]
    iota = lax.iota(jnp.int32, 16)

    def sample_loop(j, carry):
        rs = jnp.max(jnp.where(iota == 2 * j, ovec, 0))
        oy = jnp.max(jnp.where(iota == 2 * j + 1, ovec, 0))
        ay = pl.multiple_of((oy // 8) * 8, 8)
        ry = oy - ay
        ob = (wid * SPW + j) * OUT_H

        def chunk_loop(ch, carry2):
            pltpu.sync_copy(
                x2.at[pl.ds(rs + ch * R, R), pl.ds(ay, IN_W)], in_buf
            )

            def row_loop(r, carry3):
                for k in range(KV):
                    out_buf[r, pl.ds(16 * k, 16)] = in_buf[r, pl.ds(ry + 16 * k, 16)]
                return carry3

            lax.fori_loop(0, R, row_loop, 0)
            pltpu.sync_copy(out_buf, out2.at[pl.ds(ob + ch * R, R), :])
            return carry2

        lax.fori_loop(0, N_CH, chunk_loop, 0)
        return carry

    lax.fori_loop(0, SPW, sample_loop, 0)


def kernel(x):
    B, C, _, _ = x.shape
    xo, yo = _offsets(H - OUT_H, W - OUT_W)
    rowstart = jnp.arange(B_TOTAL, dtype=jnp.int32) * H + xo
    packed = jnp.stack(
        [rowstart.reshape(NW, SPW), yo.reshape(NW, SPW)], axis=-1
    ).reshape(NW, 2 * SPW)
    offs = jnp.pad(packed, ((0, 0), (0, 16 - 2 * SPW)))

    x2 = x.reshape(B * H, W)
    mesh = plsc.VectorSubcoreMesh(core_axis_name="c", subcore_axis_name="s")
    run = functools.partial(
        pl.kernel,
        out_type=jax.ShapeDtypeStruct((B * OUT_H, OUT_W), x.dtype),
        mesh=mesh,
        scratch_types=[
            pltpu.VMEM((16,), jnp.int32),
            pltpu.VMEM((R, IN_W), x.dtype),
            pltpu.VMEM((R, OUT_W), x.dtype),
        ],
        compiler_params=pltpu.CompilerParams(
            use_tc_tiling_on_sc=False, needs_layout_passes=False
        ),
    )(_sc_body)
    out2 = run(x2, offs)
    return out2.reshape(B, C, OUT_H, OUT_W)


# speedup vs baseline: 1.2812x; 1.2812x over previous
import functools

import jax
import jax.numpy as jnp
from jax import lax
from jax.experimental import pallas as pl
from jax.experimental.pallas import tpu as pltpu
from jax.experimental.pallas import tpu_sc as plsc

OUT_H, OUT_W = 448, 448
B_TOTAL = 128
H, W = 512, 512

NC, NS = 2, 16
NW = NC * NS
SPW = B_TOTAL // NW
R = 56
N_CH = OUT_H // R
NT = SPW * N_CH
KV = OUT_W // 16
IN_W = OUT_W + 8


def _offsets(h_range, w_range):
    kk = jax.random.key(42)
    kx, ky = jax.random.split(kk)
    xo = jax.random.randint(kx, (B_TOTAL,), 0, h_range, dtype=jnp.int32)
    yo = jax.random.randint(ky, (B_TOTAL,), 0, w_range, dtype=jnp.int32)
    return xo, yo


def _sc_body(x2, offs, out2, off_v, ib0, ib1, ob0, ob1, si0, si1, so0, so1):
    c = lax.axis_index("c")
    s = lax.axis_index("s")
    wid = s * NC + c
    pltpu.sync_copy(offs.at[wid], off_v)
    ovec = off_v[...]
    iota = lax.iota(jnp.int32, 16)
    ibufs, obufs = (ib0, ib1), (ob0, ob1)
    sins, souts = (si0, si1), (so0, so1)

    def params(t):
        j = t // N_CH
        ch = t - j * N_CH
        rs = jnp.max(jnp.where(iota == 2 * j, ovec, 0))
        oy = jnp.max(jnp.where(iota == 2 * j + 1, ovec, 0))
        ay = pl.multiple_of((oy // 8) * 8, 8)
        ry = oy - ay
        src = rs + ch * R
        dst = (wid * SPW + j) * OUT_H + ch * R
        return src, ay, ry, dst

    def in_copy(t, sl):
        src, ay, _, _ = params(t)
        return pltpu.make_async_copy(
            x2.at[pl.ds(src, R), pl.ds(ay, IN_W)], ibufs[sl], sins[sl]
        )

    def out_copy(t, sl):
        _, _, _, dst = params(t)
        return pltpu.make_async_copy(
            obufs[sl], out2.at[pl.ds(dst, R), :], souts[sl]
        )

    def step(t, sl):
        ib, ob = ibufs[sl], obufs[sl]
        _, _, ry, _ = params(t)
        in_copy(t, sl).wait()

        @pl.when(t >= 2)
        def _():
            out_copy(t - 2, sl).wait()

        def row_loop(r, carry):
            for k in range(KV):
                ob[r, pl.ds(16 * k, 16)] = ib[r, pl.ds(ry + 16 * k, 16)]
            return carry

        lax.fori_loop(0, R, row_loop, 0)

        @pl.when(t + 2 < NT)
        def _():
            in_copy(t + 2, sl).start()

        out_copy(t, sl).start()

    in_copy(0, 0).start()
    in_copy(1, 1).start()

    def g_loop(g, carry):
        step(2 * g, 0)
        step(2 * g + 1, 1)
        return carry

    lax.fori_loop(0, NT // 2, g_loop, 0)

    out_copy(NT - 2, 0).wait()
    out_copy(NT - 1, 1).wait()


def kernel(x):
    B, C, _, _ = x.shape
    xo, yo = _offsets(H - OUT_H, W - OUT_W)
    rowstart = jnp.arange(B_TOTAL, dtype=jnp.int32) * H + xo
    packed = jnp.stack(
        [rowstart.reshape(NW, SPW), yo.reshape(NW, SPW)], axis=-1
    ).reshape(NW, 2 * SPW)
    offs = jnp.pad(packed, ((0, 0), (0, 16 - 2 * SPW)))

    x2 = x.reshape(B * H, W)
    mesh = plsc.VectorSubcoreMesh(core_axis_name="c", subcore_axis_name="s")
    run = functools.partial(
        pl.kernel,
        out_type=jax.ShapeDtypeStruct((B * OUT_H, OUT_W), x.dtype),
        mesh=mesh,
        scratch_types=[
            pltpu.VMEM((16,), jnp.int32),
            pltpu.VMEM((R, IN_W), x.dtype),
            pltpu.VMEM((R, IN_W), x.dtype),
            pltpu.VMEM((R, OUT_W), x.dtype),
            pltpu.VMEM((R, OUT_W), x.dtype),
            pltpu.SemaphoreType.DMA,
            pltpu.SemaphoreType.DMA,
            pltpu.SemaphoreType.DMA,
            pltpu.SemaphoreType.DMA,
        ],
        compiler_params=pltpu.CompilerParams(
            use_tc_tiling_on_sc=False, needs_layout_passes=False
        ),
    )(_sc_body)
    out2 = run(x2, offs)
    return out2.reshape(B, C, OUT_H, OUT_W)


# speedup vs baseline: 3.0433x; 2.3753x over previous
import jax
import jax.numpy as jnp
import numpy as np
from jax.experimental import pallas as pl
from jax.experimental.pallas import tpu as pltpu

OUT_H, OUT_W = 448, 448
B_TOTAL = 128


def _offsets(h_range, w_range):
    kk = jax.random.key(42)
    kx, ky = jax.random.split(kk)
    xo = jax.random.randint(kx, (B_TOTAL,), 0, h_range, dtype=jnp.int32)
    yo = jax.random.randint(ky, (B_TOTAL,), 0, w_range, dtype=jnp.int32)
    return xo, yo


BLK_B = 8


def _crop_body(xo_ref, yo_ref, x_ref, o_ref):
    g = pl.program_id(0)
    for i in range(BLK_B):
        b = g * BLK_B + i
        img = x_ref[i, 0]
        img = pltpu.roll(img, -xo_ref[b], 0)
        img = pltpu.roll(img, -yo_ref[b], 1)
        o_ref[i, 0] = img[:OUT_H, :OUT_W]


def kernel(x):
    B, C, H, W = x.shape
    grid_spec = pltpu.PrefetchScalarGridSpec(
        num_scalar_prefetch=2,
        grid=(B // BLK_B,),
        in_specs=[
            pl.BlockSpec((BLK_B, 1, H, W), lambda b, xo, yo: (b, 0, 0, 0)),
        ],
        out_specs=pl.BlockSpec(
            (BLK_B, 1, OUT_H, OUT_W), lambda b, xo, yo: (b, 0, 0, 0)
        ),
    )
    xo, yo = _offsets(H - OUT_H, W - OUT_W)
    out = pl.pallas_call(
        _crop_body,
        grid_spec=grid_spec,
        out_shape=jax.ShapeDtypeStruct((B, C, OUT_H, OUT_W), x.dtype),
    )(xo, yo, x)
    return out
